# baseline (speedup 1.0000x reference)
import functools

import jax
import jax.numpy as jnp
from jax import lax
from jax.experimental import pallas as pl
from jax.experimental.pallas import tpu as pltpu
from jax.experimental.pallas import tpu_sc as plsc

B = 4096
L = 200
D = 64
NC = 2
NS = 16
NW = NC * NS
BPW = B // NW
LANES = 16
NCH = D // LANES


def _pool_sc(x, emb):
    mesh = plsc.VectorSubcoreMesh(core_axis_name="c", subcore_axis_name="s")

    @functools.partial(
        pl.kernel,
        mesh=mesh,
        out_type=jax.ShapeDtypeStruct((B, 2 * D), jnp.float32),
        scratch_types=[
            pltpu.VMEM((BPW, L), jnp.int32),
            pltpu.VMEM((L, D), jnp.float32),
            pltpu.VMEM((BPW, 2 * D), jnp.float32),
            pltpu.SemaphoreType.DMA,
        ],
        compiler_params=pltpu.CompilerParams(use_tc_tiling_on_sc=False),
    )
    def k(x_hbm, emb_hbm, out_hbm, idx_v, rows_v, out_v, sem):
        wid = lax.axis_index("s") * NC + lax.axis_index("c")
        base = wid * BPW
        pltpu.sync_copy(x_hbm.at[pl.ds(base, BPW)], idx_v)

        def row_body(r, carry):
            cp0 = pltpu.make_async_copy(
                emb_hbm.at[idx_v.at[r, pl.ds(0, 128)]],
                rows_v.at[pl.ds(0, 128)], sem)
            cp0.start()
            cp1 = pltpu.make_async_copy(
                emb_hbm.at[idx_v.at[r, pl.ds(128, L - 128)]],
                rows_v.at[pl.ds(128, L - 128)], sem)
            cp1.start()
            cp0.wait()
            cp1.wait()

            neg = jnp.full((LANES,), -3.4e38, dtype=jnp.float32)
            zero = jnp.zeros((LANES,), dtype=jnp.float32)
            init = (zero, zero, zero, zero, neg, neg, neg, neg)

            def red_body(i, acc):
                s0, s1, s2, s3, m0, m1, m2, m3 = acc
                v0 = rows_v[i, pl.ds(0, LANES)]
                v1 = rows_v[i, pl.ds(LANES, LANES)]
                v2 = rows_v[i, pl.ds(2 * LANES, LANES)]
                v3 = rows_v[i, pl.ds(3 * LANES, LANES)]
                return (s0 + v0, s1 + v1, s2 + v2, s3 + v3,
                        jnp.maximum(m0, v0), jnp.maximum(m1, v1),
                        jnp.maximum(m2, v2), jnp.maximum(m3, v3))

            s0, s1, s2, s3, m0, m1, m2, m3 = lax.fori_loop(
                0, L, red_body, init)
            inv = jnp.float32(1.0 / L)
            out_v[r, pl.ds(0, LANES)] = s0 * inv
            out_v[r, pl.ds(LANES, LANES)] = s1 * inv
            out_v[r, pl.ds(2 * LANES, LANES)] = s2 * inv
            out_v[r, pl.ds(3 * LANES, LANES)] = s3 * inv
            out_v[r, pl.ds(D, LANES)] = m0
            out_v[r, pl.ds(D + LANES, LANES)] = m1
            out_v[r, pl.ds(D + 2 * LANES, LANES)] = m2
            out_v[r, pl.ds(D + 3 * LANES, LANES)] = m3
            return carry

        lax.fori_loop(0, BPW, row_body, 0)
        pltpu.sync_copy(out_v, out_hbm.at[pl.ds(base, BPW)])

    return k(x, emb)


def _mlp_tc_body(p_ref, w1_ref, b1_ref, w2_ref, b2_ref, w3_ref, b3_ref,
                 o_ref):
    p = p_ref[...]
    h = lax.dot_general(p, w1_ref[...], (((1,), (1,)), ((), ())),
                        preferred_element_type=jnp.float32)
    h = jnp.maximum(h + b1_ref[...], 0.0)
    h2 = lax.dot_general(h, w2_ref[...], (((1,), (1,)), ((), ())),
                         preferred_element_type=jnp.float32)
    h2 = jnp.maximum(h2 + b2_ref[...], 0.0)
    o = lax.dot_general(h2, w3_ref[...], (((1,), (1,)), ((), ())),
                        preferred_element_type=jnp.float32)
    o_ref[...] = jax.nn.sigmoid(o + b3_ref[...])


def _mlp_tc(pooled, W1, b1, W2, b2, W3, b3):
    W3p = jnp.zeros((128, 64), jnp.float32).at[0].set(W3[0])
    b3p = jnp.zeros((1, 128), jnp.float32).at[0, 0].set(b3[0])
    out = pl.pallas_call(
        _mlp_tc_body,
        out_shape=jax.ShapeDtypeStruct((B, 128), jnp.float32),
    )(pooled, W1, b1.reshape(1, 128), W2, b2.reshape(1, 64), W3p, b3p)
    return out[:, :1]


def kernel(x, emb, W1, b1, W2, b2, W3, b3):
    pooled = _pool_sc(x.astype(jnp.int32), emb)
    return _mlp_tc(pooled, W1, b1, W2, b2, W3, b3)


# speedup vs baseline: 1.1771x; 1.1771x over previous
import functools

import jax
import jax.numpy as jnp
from jax import lax
from jax.experimental import pallas as pl
from jax.experimental.pallas import tpu as pltpu
from jax.experimental.pallas import tpu_sc as plsc

B = 4096
L = 200
D = 64
NC = 2
NS = 16
NW = NC * NS
BPW = B // NW
LANES = 16
NCH = D // LANES


def _pool_sc(x, emb):
    mesh = plsc.VectorSubcoreMesh(core_axis_name="c", subcore_axis_name="s")

    @functools.partial(
        pl.kernel,
        mesh=mesh,
        out_type=jax.ShapeDtypeStruct((B, 2 * D), jnp.float32),
        scratch_types=[
            pltpu.VMEM((BPW, L), jnp.int32),
            pltpu.VMEM((2, L, D), jnp.float32),
            pltpu.VMEM((BPW, 2 * D), jnp.float32),
            pltpu.SemaphoreType.DMA,
            pltpu.SemaphoreType.DMA,
        ],
        compiler_params=pltpu.CompilerParams(use_tc_tiling_on_sc=False),
    )
    def k(x_hbm, emb_hbm, out_hbm, idx_v, rows_v, out_v, sem0, sem1):
        wid = lax.axis_index("s") * NC + lax.axis_index("c")
        base = wid * BPW
        pltpu.sync_copy(x_hbm.at[pl.ds(base, BPW)], idx_v)
        sems = (sem0, sem1)

        def start_gather(row, buf):
            sem = sems[buf]
            pltpu.make_async_copy(
                emb_hbm.at[idx_v.at[row, pl.ds(0, 128)]],
                rows_v.at[buf, pl.ds(0, 128)], sem).start()
            pltpu.make_async_copy(
                emb_hbm.at[idx_v.at[row, pl.ds(128, L - 128)]],
                rows_v.at[buf, pl.ds(128, L - 128)], sem).start()

        def wait_gather(buf):
            sem = sems[buf]
            pltpu.make_async_copy(
                emb_hbm.at[pl.ds(0, 128)], rows_v.at[buf, pl.ds(0, 128)],
                sem).wait()
            pltpu.make_async_copy(
                emb_hbm.at[pl.ds(0, L - 128)],
                rows_v.at[buf, pl.ds(128, L - 128)], sem).wait()

        def reduce_row(row, buf):
            neg = jnp.full((LANES,), -3.4e38, dtype=jnp.float32)
            zero = jnp.zeros((LANES,), dtype=jnp.float32)
            init = (zero, zero, zero, zero, neg, neg, neg, neg)

            def red_body(i, acc):
                s0, s1, s2, s3, m0, m1, m2, m3 = acc
                v0 = rows_v[buf, i, pl.ds(0, LANES)]
                v1 = rows_v[buf, i, pl.ds(LANES, LANES)]
                v2 = rows_v[buf, i, pl.ds(2 * LANES, LANES)]
                v3 = rows_v[buf, i, pl.ds(3 * LANES, LANES)]
                return (s0 + v0, s1 + v1, s2 + v2, s3 + v3,
                        jnp.maximum(m0, v0), jnp.maximum(m1, v1),
                        jnp.maximum(m2, v2), jnp.maximum(m3, v3))

            s0, s1, s2, s3, m0, m1, m2, m3 = lax.fori_loop(
                0, L, red_body, init, unroll=4)
            inv = jnp.float32(1.0 / L)
            out_v[row, pl.ds(0, LANES)] = s0 * inv
            out_v[row, pl.ds(LANES, LANES)] = s1 * inv
            out_v[row, pl.ds(2 * LANES, LANES)] = s2 * inv
            out_v[row, pl.ds(3 * LANES, LANES)] = s3 * inv
            out_v[row, pl.ds(D, LANES)] = m0
            out_v[row, pl.ds(D + LANES, LANES)] = m1
            out_v[row, pl.ds(D + 2 * LANES, LANES)] = m2
            out_v[row, pl.ds(D + 3 * LANES, LANES)] = m3

        start_gather(0, 0)
        start_gather(1, 1)

        def pair_body(i, carry):
            row = 2 * i
            for buf in (0, 1):
                wait_gather(buf)
                reduce_row(row + buf, buf)

                @pl.when(row + buf + 2 < BPW)
                def _():
                    start_gather(row + buf + 2, buf)
            return carry

        lax.fori_loop(0, BPW // 2, pair_body, 0)
        pltpu.sync_copy(out_v, out_hbm.at[pl.ds(base, BPW)])

    return k(x, emb)


def _mlp_tc_body(p_ref, w1_ref, b1_ref, w2_ref, b2_ref, w3_ref, b3_ref,
                 o_ref):
    p = p_ref[...]
    h = lax.dot_general(p, w1_ref[...], (((1,), (1,)), ((), ())),
                        preferred_element_type=jnp.float32)
    h = jnp.maximum(h + b1_ref[...], 0.0)
    h2 = lax.dot_general(h, w2_ref[...], (((1,), (1,)), ((), ())),
                         preferred_element_type=jnp.float32)
    h2 = jnp.maximum(h2 + b2_ref[...], 0.0)
    o = lax.dot_general(h2, w3_ref[...], (((1,), (1,)), ((), ())),
                        preferred_element_type=jnp.float32)
    o_ref[...] = jax.nn.sigmoid(o + b3_ref[...])


def _mlp_tc(pooled, W1, b1, W2, b2, W3, b3):
    W3p = jnp.zeros((128, 64), jnp.float32).at[0].set(W3[0])
    b3p = jnp.zeros((1, 128), jnp.float32).at[0, 0].set(b3[0])
    out = pl.pallas_call(
        _mlp_tc_body,
        out_shape=jax.ShapeDtypeStruct((B, 128), jnp.float32),
    )(pooled, W1, b1.reshape(1, 128), W2, b2.reshape(1, 64), W3p, b3p)
    return out[:, :1]


def kernel(x, emb, W1, b1, W2, b2, W3, b3):
    pooled = _pool_sc(x.astype(jnp.int32), emb)
    return _mlp_tc(pooled, W1, b1, W2, b2, W3, b3)
